# baseline (speedup 1.0000x reference)
import functools
import jax
import jax.numpy as jnp
from jax import lax
from jax.experimental import pallas as pl
from jax.experimental.pallas import tpu as pltpu
from jax.experimental.pallas import tpu_sc as plsc

_D = 768
_L = 16
_NV = _D // _L
_B = 16
_N0, _N1, _N2 = 1024, 256, 64
_NW = 32
_R0, _R1, _R2 = _N0 // _NW, _N1 // _NW, _N2 // _NW
_RT = _R0 + _R1 + _R2


def _sc_body(f0_hbm, f1_hbm, f2_hbm, se_hbm, pe_hbm, out_hbm,
             buf, tbl, se_v):
    wid = lax.axis_index("s") * 2 + lax.axis_index("c")
    r0 = _R0 * wid
    r1 = _R1 * wid
    r2 = _R2 * wid

    pltpu.sync_copy(pe_hbm.at[0, pl.ds(r0, _R0)], tbl.at[pl.ds(0, _R0)])
    pltpu.sync_copy(pe_hbm.at[1, pl.ds(r1, _R1)], tbl.at[pl.ds(_R0, _R1)])
    pltpu.sync_copy(pe_hbm.at[2, pl.ds(r2, _R2)], tbl.at[pl.ds(_R0 + _R1, _R2)])
    pltpu.sync_copy(se_hbm, se_v)

    def _add_se(base, nrows, s):
        def row(r, carry):
            for k in range(_NV):
                sl = pl.ds(k * _L, _L)
                tbl[base + r, sl] = tbl[base + r, sl] + se_v[s, sl]
            return carry
        lax.fori_loop(0, nrows, row, 0)

    _add_se(0, _R0, 0)
    _add_se(_R0, _R1, 1)
    _add_se(_R0 + _R1, _R2, 2)

    def batch(b, carry):
        pltpu.sync_copy(f0_hbm.at[b, pl.ds(r0, _R0)], buf.at[pl.ds(0, _R0)])
        pltpu.sync_copy(f1_hbm.at[b, pl.ds(r1, _R1)], buf.at[pl.ds(_R0, _R1)])
        pltpu.sync_copy(f2_hbm.at[b, pl.ds(r2, _R2)],
                        buf.at[pl.ds(_R0 + _R1, _R2)])

        def row(r, c2):
            for k in range(_NV):
                sl = pl.ds(k * _L, _L)
                plsc.addupdate(buf.at[r, sl], tbl[r, sl])
            return c2
        lax.fori_loop(0, _RT, row, 0)

        pltpu.sync_copy(buf.at[pl.ds(0, _R0)],
                        out_hbm.at[b, pl.ds(r0, _R0)])
        pltpu.sync_copy(buf.at[pl.ds(_R0, _R1)],
                        out_hbm.at[b, pl.ds(_N0 + r1, _R1)])
        pltpu.sync_copy(buf.at[pl.ds(_R0 + _R1, _R2)],
                        out_hbm.at[b, pl.ds(_N0 + _N1 + r2, _R2)])
        return carry
    lax.fori_loop(0, _B, batch, 0)


def kernel(features_per_scale_0, features_per_scale_1, features_per_scale_2,
           scale_embeddings, patch_embeddings):
    mesh = plsc.VectorSubcoreMesh(core_axis_name="c", subcore_axis_name="s")
    ker = pl.kernel(
        _sc_body,
        out_type=jax.ShapeDtypeStruct((_B, _N0 + _N1 + _N2, _D), jnp.float32),
        mesh=mesh,
        scratch_types=[
            pltpu.VMEM((_RT, _D), jnp.float32),
            pltpu.VMEM((_RT, _D), jnp.float32),
            pltpu.VMEM((3, _D), jnp.float32),
        ],
    )
    return ker(features_per_scale_0, features_per_scale_1,
               features_per_scale_2, scale_embeddings, patch_embeddings)


# speedup vs baseline: 1.1629x; 1.1629x over previous
import jax
import jax.numpy as jnp
from jax import lax
from jax.experimental import pallas as pl
from jax.experimental.pallas import tpu as pltpu
from jax.experimental.pallas import tpu_sc as plsc

_D = 768
_L = 16
_NV = _D // _L
_B = 16
_N0, _N1, _N2 = 1024, 256, 64
_NW = 32
_R0, _R1, _R2 = _N0 // _NW, _N1 // _NW, _N2 // _NW
_RT = _R0 + _R1 + _R2


def _sc_body(f0_hbm, f1_hbm, f2_hbm, se_hbm, pe_hbm, out_hbm,
             buf, tbl, se_v, in_sems, out_sems):
    wid = lax.axis_index("s") * 2 + lax.axis_index("c")
    r0 = _R0 * wid
    r1 = _R1 * wid
    r2 = _R2 * wid

    pltpu.sync_copy(pe_hbm.at[0, pl.ds(r0, _R0)], tbl.at[pl.ds(0, _R0)])
    pltpu.sync_copy(pe_hbm.at[1, pl.ds(r1, _R1)], tbl.at[pl.ds(_R0, _R1)])
    pltpu.sync_copy(pe_hbm.at[2, pl.ds(r2, _R2)], tbl.at[pl.ds(_R0 + _R1, _R2)])
    pltpu.sync_copy(se_hbm, se_v)

    def _add_se(base, nrows, s):
        def row(r, carry):
            for k in range(_NV):
                sl = pl.ds(k * _L, _L)
                tbl[base + r, sl] = tbl[base + r, sl] + se_v[s, sl]
            return carry
        lax.fori_loop(0, nrows, row, 0)

    _add_se(0, _R0, 0)
    _add_se(_R0, _R1, 1)
    _add_se(_R0 + _R1, _R2, 2)

    def start_in(b, p):
        return (
            pltpu.async_copy(f0_hbm.at[b, pl.ds(r0, _R0)],
                             buf.at[p, pl.ds(0, _R0)], in_sems.at[p]),
            pltpu.async_copy(f1_hbm.at[b, pl.ds(r1, _R1)],
                             buf.at[p, pl.ds(_R0, _R1)], in_sems.at[p]),
            pltpu.async_copy(f2_hbm.at[b, pl.ds(r2, _R2)],
                             buf.at[p, pl.ds(_R0 + _R1, _R2)], in_sems.at[p]),
        )

    def start_out(b, p):
        return (
            pltpu.async_copy(buf.at[p, pl.ds(0, _R0)],
                             out_hbm.at[b, pl.ds(r0, _R0)], out_sems.at[p]),
            pltpu.async_copy(buf.at[p, pl.ds(_R0, _R1)],
                             out_hbm.at[b, pl.ds(_N0 + r1, _R1)],
                             out_sems.at[p]),
            pltpu.async_copy(buf.at[p, pl.ds(_R0 + _R1, _R2)],
                             out_hbm.at[b, pl.ds(_N0 + _N1 + r2, _R2)],
                             out_sems.at[p]),
        )

    def add_table(p):
        def row(r, c2):
            for k in range(_NV):
                sl = pl.ds(k * _L, _L)
                plsc.addupdate(buf.at[p, r, sl], tbl[r, sl])
            return c2
        lax.fori_loop(0, _RT, row, 0)

    pending_out = [None, None]
    pending_in = [None, None]
    pending_in[0] = start_in(0, 0)
    for b in range(_B):
        p = b % 2
        if b + 1 < _B:
            if pending_out[1 - p] is not None:
                for h in pending_out[1 - p]:
                    h.wait()
            pending_in[1 - p] = start_in(b + 1, 1 - p)
        for h in pending_in[p]:
            h.wait()
        add_table(p)
        pending_out[p] = start_out(b, p)
    for p in range(2):
        if pending_out[p] is not None:
            for h in pending_out[p]:
                h.wait()


def kernel(features_per_scale_0, features_per_scale_1, features_per_scale_2,
           scale_embeddings, patch_embeddings):
    mesh = plsc.VectorSubcoreMesh(core_axis_name="c", subcore_axis_name="s")
    ker = pl.kernel(
        _sc_body,
        out_type=jax.ShapeDtypeStruct((_B, _N0 + _N1 + _N2, _D), jnp.float32),
        mesh=mesh,
        scratch_types=[
            pltpu.VMEM((2, _RT, _D), jnp.float32),
            pltpu.VMEM((_RT, _D), jnp.float32),
            pltpu.VMEM((3, _D), jnp.float32),
            pltpu.SemaphoreType.DMA((2,)),
            pltpu.SemaphoreType.DMA((2,)),
        ],
    )
    return ker(features_per_scale_0, features_per_scale_1,
               features_per_scale_2, scale_embeddings, patch_embeddings)


# speedup vs baseline: 3.9126x; 3.3645x over previous
import jax
import jax.numpy as jnp
from jax.experimental import pallas as pl
from jax.experimental.pallas import tpu as pltpu

_D = 768
_N0, _N1, _N2 = 1024, 256, 64
_NTOT = _N0 + _N1 + _N2


_BB = 2


def _body(f0_ref, f1_ref, f2_ref, se_ref, pe0_ref, pe1_ref, pe2_ref, out_ref):
    out_ref[:, 0:_N0, :] = (
        f0_ref[...] + (se_ref[0, :][None, None, :] + pe0_ref[...]))
    out_ref[:, _N0:_N0 + _N1, :] = (
        f1_ref[...] + (se_ref[1, :][None, None, :] + pe1_ref[...]))
    out_ref[:, _N0 + _N1:_NTOT, :] = (
        f2_ref[...] + (se_ref[2, :][None, None, :] + pe2_ref[...]))


def kernel(features_per_scale_0, features_per_scale_1, features_per_scale_2,
           scale_embeddings, patch_embeddings):
    B = features_per_scale_0.shape[0]

    return pl.pallas_call(
        _body,
        grid=(B // _BB,),
        in_specs=[
            pl.BlockSpec((_BB, _N0, _D), lambda b: (b, 0, 0)),
            pl.BlockSpec((_BB, _N1, _D), lambda b: (b, 0, 0)),
            pl.BlockSpec((_BB, _N2, _D), lambda b: (b, 0, 0)),
            pl.BlockSpec((3, _D), lambda b: (0, 0)),
            pl.BlockSpec((1, _N0, _D), lambda b: (0, 0, 0)),
            pl.BlockSpec((1, _N1, _D), lambda b: (1, 0, 0)),
            pl.BlockSpec((1, _N2, _D), lambda b: (2, 0, 0)),
        ],
        out_specs=pl.BlockSpec((_BB, _NTOT, _D), lambda b: (b, 0, 0)),
        out_shape=jax.ShapeDtypeStruct((B, _NTOT, _D), jnp.float32),
        compiler_params=pltpu.CompilerParams(
            dimension_semantics=("parallel",),
            vmem_limit_bytes=120 * 1024 * 1024),
    )(features_per_scale_0, features_per_scale_1, features_per_scale_2,
      scale_embeddings, patch_embeddings, patch_embeddings, patch_embeddings)


# speedup vs baseline: 3.9203x; 1.0020x over previous
import jax
import jax.numpy as jnp
from jax.experimental import pallas as pl
from jax.experimental.pallas import tpu as pltpu

_D = 768
_N0, _N1, _N2 = 1024, 256, 64
_NTOT = _N0 + _N1 + _N2


_BB = 2


def _body(f0_ref, f1_ref, f2_ref, se_ref, pe0_ref, pe1_ref, pe2_ref, out_ref):
    out_ref[:, 0:_N0, :] = (
        f0_ref[...] + (se_ref[0, :][None, None, :] + pe0_ref[...]))
    out_ref[:, _N0:_N0 + _N1, :] = (
        f1_ref[...] + (se_ref[1, :][None, None, :] + pe1_ref[...]))
    out_ref[:, _N0 + _N1:_NTOT, :] = (
        f2_ref[...] + (se_ref[2, :][None, None, :] + pe2_ref[...]))


def kernel(features_per_scale_0, features_per_scale_1, features_per_scale_2,
           scale_embeddings, patch_embeddings):
    B = features_per_scale_0.shape[0]

    return pl.pallas_call(
        _body,
        grid=(B // _BB,),
        in_specs=[
            pl.BlockSpec((_BB, _N0, _D), lambda b: (b, 0, 0)),
            pl.BlockSpec((_BB, _N1, _D), lambda b: (b, 0, 0)),
            pl.BlockSpec((_BB, _N2, _D), lambda b: (b, 0, 0)),
            pl.BlockSpec((3, _D), lambda b: (0, 0)),
            pl.BlockSpec((1, _N0, _D), lambda b: (0, 0, 0)),
            pl.BlockSpec((1, _N1, _D), lambda b: (1, 0, 0)),
            pl.BlockSpec((1, _N2, _D), lambda b: (2, 0, 0)),
        ],
        out_specs=pl.BlockSpec((_BB, _NTOT, _D), lambda b: (b, 0, 0)),
        out_shape=jax.ShapeDtypeStruct((B, _NTOT, _D), jnp.float32),
        compiler_params=pltpu.CompilerParams(
            dimension_semantics=("arbitrary",),
            vmem_limit_bytes=120 * 1024 * 1024),
    )(features_per_scale_0, features_per_scale_1, features_per_scale_2,
      scale_embeddings, patch_embeddings, patch_embeddings, patch_embeddings)
